# Initial kernel scaffold; baseline (speedup 1.0000x reference)
import functools

import jax
import jax.numpy as jnp
from jax import lax
from jax.experimental import pallas as pl
from jax.experimental.pallas import tpu as pltpu
from jax.experimental.pallas import tpu_sc as plsc

NUM_MOVES = 1000
EMBED_DIM = 64
BATCH = 16384
NUM_POKEMON = 6
NUM_MOVE_SLOTS = 4

TOTAL = BATCH * NUM_POKEMON * NUM_MOVE_SLOTS

NUM_CORES = 2
NUM_SUBCORES = 16
NUM_WORKERS = NUM_CORES * NUM_SUBCORES

CHUNK = 128
ROWS_PER_WORKER = TOTAL // NUM_WORKERS
CHUNKS_PER_WORKER = ROWS_PER_WORKER // CHUNK


@functools.partial(
    pl.kernel,
    out_type=jax.ShapeDtypeStruct((TOTAL, EMBED_DIM), jnp.float32),
    mesh=plsc.VectorSubcoreMesh(core_axis_name="c", subcore_axis_name="s"),
    scratch_types=[
        pltpu.VMEM((CHUNKS_PER_WORKER, CHUNK), jnp.int32),
        pltpu.VMEM((CHUNK, EMBED_DIM), jnp.float32),
        pltpu.SemaphoreType.DMA,
    ],
)
def _gather_kernel(idx_hbm, table_hbm, out_hbm, idx_v, rows_v, sem):
    wid = lax.axis_index("s") * NUM_CORES + lax.axis_index("c")
    row_base = wid * ROWS_PER_WORKER
    chunk_base = wid * CHUNKS_PER_WORKER
    pltpu.sync_copy(idx_hbm.at[pl.ds(chunk_base, CHUNKS_PER_WORKER)], idx_v)

    def body(j, _):
        pltpu.async_copy(table_hbm.at[idx_v.at[j]], rows_v, sem).wait()
        pltpu.sync_copy(
            rows_v, out_hbm.at[pl.ds(row_base + j * CHUNK, CHUNK)])
        return ()

    lax.fori_loop(0, CHUNKS_PER_WORKER, body, ())


def kernel(move_ids, table):
    idx = move_ids.reshape(TOTAL // CHUNK, CHUNK).astype(jnp.int32)
    out = _gather_kernel(idx, table)
    return out.reshape(BATCH, NUM_POKEMON, NUM_MOVE_SLOTS, EMBED_DIM)


# speedup vs baseline: 2.8969x; 2.8969x over previous
import functools

import jax
import jax.numpy as jnp
from jax import lax
from jax.experimental import pallas as pl
from jax.experimental.pallas import tpu as pltpu
from jax.experimental.pallas import tpu_sc as plsc

NUM_MOVES = 1000
EMBED_DIM = 64
BATCH = 16384
NUM_POKEMON = 6
NUM_MOVE_SLOTS = 4

TOTAL = BATCH * NUM_POKEMON * NUM_MOVE_SLOTS

NUM_CORES = 2
NUM_SUBCORES = 16
NUM_WORKERS = NUM_CORES * NUM_SUBCORES

CHUNK = 128
ROWS_PER_WORKER = TOTAL // NUM_WORKERS
CHUNKS_PER_WORKER = ROWS_PER_WORKER // CHUNK


@functools.partial(
    pl.kernel,
    out_type=jax.ShapeDtypeStruct((TOTAL, EMBED_DIM), jnp.float32),
    mesh=plsc.VectorSubcoreMesh(core_axis_name="c", subcore_axis_name="s"),
    compiler_params=pltpu.CompilerParams(use_tc_tiling_on_sc=False),
    scratch_types=[
        pltpu.VMEM((CHUNKS_PER_WORKER, CHUNK), jnp.int32),
        pltpu.VMEM((CHUNK, EMBED_DIM), jnp.float32),
        pltpu.SemaphoreType.DMA,
    ],
)
def _gather_kernel(idx_hbm, table_hbm, out_hbm, idx_v, rows_v, sem):
    wid = lax.axis_index("s") * NUM_CORES + lax.axis_index("c")
    row_base = wid * ROWS_PER_WORKER
    chunk_base = wid * CHUNKS_PER_WORKER
    pltpu.sync_copy(idx_hbm.at[pl.ds(chunk_base, CHUNKS_PER_WORKER)], idx_v)

    def body(j, _):
        pltpu.async_copy(table_hbm.at[idx_v.at[j]], rows_v, sem).wait()
        pltpu.sync_copy(
            rows_v, out_hbm.at[pl.ds(row_base + j * CHUNK, CHUNK)])
        return ()

    lax.fori_loop(0, CHUNKS_PER_WORKER, body, ())


def kernel(move_ids, table):
    idx = move_ids.reshape(TOTAL // CHUNK, CHUNK).astype(jnp.int32)
    out = _gather_kernel(idx, table)
    return out.reshape(BATCH, NUM_POKEMON, NUM_MOVE_SLOTS, EMBED_DIM)


# speedup vs baseline: 3.0170x; 1.0415x over previous
import functools

import jax
import jax.numpy as jnp
from jax import lax
from jax.experimental import pallas as pl
from jax.experimental.pallas import tpu as pltpu
from jax.experimental.pallas import tpu_sc as plsc

NUM_MOVES = 1000
EMBED_DIM = 64
BATCH = 16384
NUM_POKEMON = 6
NUM_MOVE_SLOTS = 4

TOTAL = BATCH * NUM_POKEMON * NUM_MOVE_SLOTS

NUM_CORES = 2
NUM_SUBCORES = 16
NUM_WORKERS = NUM_CORES * NUM_SUBCORES

CHUNK = 128
ROWS_PER_WORKER = TOTAL // NUM_WORKERS
CHUNKS_PER_WORKER = ROWS_PER_WORKER // CHUNK

NBUF = 8
DEPTH = 4
STEPS = CHUNKS_PER_WORKER // NBUF

CHUNK_BYTES = CHUNK * EMBED_DIM * 4


@functools.partial(
    pl.kernel,
    out_type=jax.ShapeDtypeStruct((TOTAL, EMBED_DIM), jnp.float32),
    mesh=plsc.VectorSubcoreMesh(core_axis_name="c", subcore_axis_name="s"),
    compiler_params=pltpu.CompilerParams(use_tc_tiling_on_sc=False),
    scratch_types=[
        pltpu.VMEM((CHUNKS_PER_WORKER, CHUNK), jnp.int32),
        pltpu.VMEM((NBUF, CHUNK, EMBED_DIM), jnp.float32),
        [pltpu.SemaphoreType.DMA] * NBUF,
        [pltpu.SemaphoreType.DMA] * NBUF,
    ],
)
def _gather_kernel(idx_hbm, table_hbm, out_hbm, idx_v, rows_v, g_sems,
                   o_sems):
    wid = lax.axis_index("s") * NUM_CORES + lax.axis_index("c")
    row_base = wid * ROWS_PER_WORKER
    chunk_base = wid * CHUNKS_PER_WORKER
    pltpu.sync_copy(idx_hbm.at[pl.ds(chunk_base, CHUNKS_PER_WORKER)], idx_v)

    def fire_gather(c, buf):
        pltpu.async_copy(table_hbm.at[idx_v.at[c]], rows_v.at[buf],
                         g_sems[buf])

    def wait_gather(c, buf):
        pltpu.make_async_copy(table_hbm.at[idx_v.at[c]], rows_v.at[buf],
                              g_sems[buf]).wait()

    def fire_writeback(c, buf):
        pltpu.async_copy(rows_v.at[buf],
                         out_hbm.at[pl.ds(row_base + c * CHUNK, CHUNK)],
                         o_sems[buf])

    def wait_writeback(buf):
        pltpu.make_async_copy(rows_v.at[buf], out_hbm.at[pl.ds(0, CHUNK)],
                              o_sems[buf]).wait()

    for b in range(DEPTH):
        fire_gather(b, b)

    def body(step, _):
        for b in range(NBUF):
            j = step * NBUF + b
            c = j + DEPTH
            pb = (b + DEPTH) % NBUF

            @pl.when(jnp.logical_and(c < CHUNKS_PER_WORKER, c >= NBUF))
            def _():
                wait_writeback(pb)

            @pl.when(c < CHUNKS_PER_WORKER)
            def _():
                fire_gather(c, pb)

            wait_gather(j, b)
            fire_writeback(j, b)
        return ()

    lax.fori_loop(0, STEPS, body, ())

    for b in range(NBUF):
        wait_writeback(b)


def kernel(move_ids, table):
    idx = move_ids.reshape(TOTAL // CHUNK, CHUNK).astype(jnp.int32)
    out = _gather_kernel(idx, table)
    return out.reshape(BATCH, NUM_POKEMON, NUM_MOVE_SLOTS, EMBED_DIM)


# speedup vs baseline: 3.3788x; 1.1199x over previous
import functools

import jax
import jax.numpy as jnp
from jax import lax
from jax.experimental import pallas as pl
from jax.experimental.pallas import tpu as pltpu
from jax.experimental.pallas import tpu_sc as plsc

NUM_MOVES = 1000
EMBED_DIM = 64
BATCH = 16384
NUM_POKEMON = 6
NUM_MOVE_SLOTS = 4

ROWS_PER_ENTRY = NUM_POKEMON * NUM_MOVE_SLOTS
TOTAL = BATCH * ROWS_PER_ENTRY

NUM_CORES = 2
NUM_SUBCORES = 16
NUM_WORKERS = NUM_CORES * NUM_SUBCORES

ENTRIES_PER_CHUNK = 4
CHUNK = ENTRIES_PER_CHUNK * ROWS_PER_ENTRY
ENTRIES_PER_WORKER = BATCH // NUM_WORKERS
CHUNKS_PER_WORKER = ENTRIES_PER_WORKER // ENTRIES_PER_CHUNK

NBUF = 8
DEPTH = 4
STEPS = CHUNKS_PER_WORKER // NBUF


@functools.partial(
    pl.kernel,
    out_type=jax.ShapeDtypeStruct((TOTAL, EMBED_DIM), jnp.float32),
    mesh=plsc.VectorSubcoreMesh(core_axis_name="c", subcore_axis_name="s"),
    compiler_params=pltpu.CompilerParams(use_tc_tiling_on_sc=False),
    scratch_types=[
        pltpu.VMEM((CHUNKS_PER_WORKER, CHUNK), jnp.int32),
        pltpu.VMEM((NBUF, CHUNK, EMBED_DIM), jnp.float32),
        [pltpu.SemaphoreType.DMA] * NBUF,
        [pltpu.SemaphoreType.DMA] * NBUF,
    ],
)
def _gather_kernel(idx_hbm, table_hbm, out_hbm, idx_v, rows_v, g_sems,
                   o_sems):
    wid = lax.axis_index("s") * NUM_CORES + lax.axis_index("c")
    row_base = wid * CHUNKS_PER_WORKER * CHUNK
    chunk_base = wid * CHUNKS_PER_WORKER
    pltpu.sync_copy(idx_hbm.at[pl.ds(chunk_base, CHUNKS_PER_WORKER)], idx_v)

    def fire_gather(c, buf):
        pltpu.async_copy(table_hbm.at[idx_v.at[c]], rows_v.at[buf],
                         g_sems[buf])

    def wait_gather(c, buf):
        pltpu.make_async_copy(table_hbm.at[idx_v.at[c]], rows_v.at[buf],
                              g_sems[buf]).wait()

    def fire_writeback(c, buf):
        pltpu.async_copy(
            rows_v.at[buf],
            out_hbm.at[pl.ds(row_base + c * CHUNK, CHUNK)],
            o_sems[buf])

    def wait_writeback(buf):
        pltpu.make_async_copy(
            rows_v.at[buf], out_hbm.at[pl.ds(0, CHUNK)],
            o_sems[buf]).wait()

    for b in range(DEPTH):
        fire_gather(b, b)

    def body(step, _):
        for b in range(NBUF):
            j = step * NBUF + b
            c = j + DEPTH
            pb = (b + DEPTH) % NBUF

            @pl.when(jnp.logical_and(c < CHUNKS_PER_WORKER, c >= NBUF))
            def _():
                wait_writeback(pb)

            @pl.when(c < CHUNKS_PER_WORKER)
            def _():
                fire_gather(c, pb)

            wait_gather(j, b)
            fire_writeback(j, b)
        return ()

    lax.fori_loop(0, STEPS, body, ())

    for b in range(NBUF):
        wait_writeback(b)


def kernel(move_ids, table):
    table_rep = jnp.tile(table, (NUM_WORKERS, 1))
    idx = move_ids.reshape(TOTAL // CHUNK, CHUNK).astype(jnp.int32)
    idx = idx + (jnp.arange(TOTAL // CHUNK, dtype=jnp.int32)[:, None]
                 // CHUNKS_PER_WORKER) * NUM_MOVES
    out = _gather_kernel(idx, table_rep)
    return out.reshape(BATCH, NUM_POKEMON, NUM_MOVE_SLOTS, EMBED_DIM)
